# baseline (speedup 1.0000x reference)
import functools

import jax
import jax.numpy as jnp
from jax import lax
from jax.experimental import pallas as pl
from jax.experimental.pallas import tpu as pltpu
from jax.experimental.pallas import tpu_sc as plsc

N = 1000
C = 51
H = 256
NT = 16
RPT = 64
NG = RPT // 16
NP = NT * RPT
BIG = 2**30

RB = 6 * C + 2

_DNUMS = jax.lax.GatherDimensionNumbers(
    offset_dims=(), collapsed_slice_dims=(0,), start_index_map=(0,))


def _g16(v, idx):
    return jax.lax.gather(v, idx[:, None], _DNUMS, (1,),
                          mode=jax.lax.GatherScatterMode.PROMISE_IN_BOUNDS)


def _bmax(v, iota):
    for sh in (8, 4, 2, 1):
        v = jnp.maximum(v, _g16(v, jnp.bitwise_xor(iota, sh)))
    return v


def _bmin(v, iota):
    for sh in (8, 4, 2, 1):
        v = jnp.minimum(v, _g16(v, jnp.bitwise_xor(iota, sh)))
    return v


def _smax(v, iota):
    for sh in (8, 4, 2):
        v = jnp.maximum(v, _g16(v, jnp.bitwise_xor(iota, sh)))
    return jnp.maximum(v[0], v[1])


def _smin(v, iota):
    for sh in (8, 4, 2):
        v = jnp.minimum(v, _g16(v, jnp.bitwise_xor(iota, sh)))
    return jnp.minimum(v[0], v[1])


def _scal(v, iota):
    return jnp.where(iota == 0, v, jnp.zeros_like(v))[0]


def _prep_kernel(d1_ref, feats_ref, wt_ref, b_ref,
                 x1t_ref, y1t_ref, x2t_ref, y2t_ref,
                 d2_ref, pt_ref, areas_ref, cmax_ref, carg_ref):
    d2 = jnp.dot(feats_ref[...], wt_ref[...],
                 preferred_element_type=jnp.float32)
    d2 = d2 + b_ref[...] + d1_ref[...]
    d2_ref[...] = d2
    d2t = jnp.transpose(d2)
    mx = jnp.max(d2t, axis=0, keepdims=True)
    e = jnp.exp(d2t - mx)
    s = jnp.sum(e, axis=0, keepdims=True)
    pt = e / s
    row = lax.broadcasted_iota(jnp.int32, (C, N), 0)
    pt = jnp.where(row == 0, 0.0, pt)
    pt_ref[...] = pt
    areas_ref[...] = ((x2t_ref[...] - x1t_ref[...]) + 1.0) * \
                     ((y2t_ref[...] - y1t_ref[...]) + 1.0)
    cm = jnp.max(pt, axis=0, keepdims=True)
    cmax_ref[...] = cm
    carg_ref[...] = jnp.min(jnp.where(pt == cm, row, BIG),
                            axis=0, keepdims=True).astype(jnp.float32)


def _sc_greedy(x_hbm, out_hbm, blk, cmax2, carg2, preds2, predsi, rec_v,
               tab_v, shared):
    tid = lax.axis_index("s")
    base = tid * RPT
    pltpu.sync_copy(x_hbm.at[tid], blk)
    iota = lax.iota(jnp.int32, 16)
    fiota = iota.astype(jnp.float32)

    for g in range(NG):
        cmax2[g, :] = blk[(RB - 2) * NG + g, :]
        carg2[g, :] = blk[(RB - 1) * NG + g, :]
        preds2[g, :] = jnp.zeros((16,), jnp.float32)

    def body(it, carry):
        sel = it & 1
        cvs = [cmax2[g, :] for g in range(NG)]
        vm = jnp.maximum(jnp.maximum(cvs[0], cvs[1]),
                         jnp.maximum(cvs[2], cvs[3]))
        m16 = _bmax(vm, iota)
        cand = jnp.full((16,), 1024.0, jnp.float32)
        for g in range(NG):
            cand = jnp.minimum(cand, jnp.where(cvs[g] == m16,
                                               fiota + (g * 16), 1024.0))
        j = _smin(cand, iota).astype(jnp.int32)
        gj = j >> 4
        lj = jnp.full((16,), j & 15, jnp.int32)
        cl16 = _g16(carg2[gj, :], lj)
        cloc = _scal(cl16, iota).astype(jnp.int32)
        x1c = _g16(blk[(C + cloc) * NG + gj, :], lj)
        y1c = _g16(blk[(2 * C + cloc) * NG + gj, :], lj)
        x2c = _g16(blk[(3 * C + cloc) * NG + gj, :], lj)
        y2c = _g16(blk[(4 * C + cloc) * NG + gj, :], lj)
        ac = _g16(blk[(5 * C + cloc) * NG + gj, :], lj)
        nf = (base + j).astype(jnp.float32)
        rec = jnp.where(iota == 0, m16, 0.0)
        rec = jnp.where(iota == 1, nf, rec)
        rec = jnp.where(iota == 2, cl16, rec)
        rec = jnp.where(iota == 3, x1c, rec)
        rec = jnp.where(iota == 4, y1c, rec)
        rec = jnp.where(iota == 5, x2c, rec)
        rec = jnp.where(iota == 6, y2c, rec)
        rec = jnp.where(iota == 7, ac, rec)
        rec_v[...] = rec
        pltpu.sync_copy(rec_v, shared.at[sel, tid])
        plsc.subcore_barrier()
        pltpu.sync_copy(shared.at[sel], tab_v)

        zero16 = jnp.zeros((16,), jnp.int32)
        one16 = jnp.full((16,), 1, jnp.int32)
        vals = jnp.zeros((16,), jnp.float32)
        nvec = jnp.zeros((16,), jnp.float32)
        for t in range(NT):
            rowt = tab_v[t, :]
            vals = jnp.where(iota == t, _g16(rowt, zero16), vals)
            nvec = jnp.where(iota == t, _g16(rowt, one16), nvec)
        gm16 = _bmax(vals, iota)
        nm = vals == gm16
        nstar16 = _bmin(jnp.where(nm, nvec, 1e9), iota)
        tcand = jnp.where(nm & (nvec == nstar16), fiota, 64.0)
        tstar = _smin(tcand, iota).astype(jnp.int32)
        wrec = tab_v[tstar, :]
        nstar16f = _g16(wrec, one16)
        cst16 = _g16(wrec, one16 + 1)
        cstar = _scal(cst16, iota).astype(jnp.int32)
        cstar_f = cst16
        X1 = _g16(wrec, one16 + 2)
        Y1 = _g16(wrec, one16 + 3)
        X2 = _g16(wrec, one16 + 4)
        Y2 = _g16(wrec, one16 + 5)
        A = _g16(wrec, one16 + 6)

        for g in range(NG):
            pv = blk[cstar * NG + g, :]
            x1v = blk[(C + cstar) * NG + g, :]
            y1v = blk[(2 * C + cstar) * NG + g, :]
            x2v = blk[(3 * C + cstar) * NG + g, :]
            y2v = blk[(4 * C + cstar) * NG + g, :]
            av = blk[(5 * C + cstar) * NG + g, :]
            iw = jnp.maximum(jnp.minimum(X2, x2v) - jnp.maximum(X1, x1v)
                             + 1.0, 0.0)
            ih = jnp.maximum(jnp.minimum(Y2, y2v) - jnp.maximum(Y1, y1v)
                             + 1.0, 0.0)
            inters = iw * ih
            union = (-inters + av) + A
            ovl = (inters / union) >= 0.5
            blk[cstar * NG + g, :] = jnp.where(ovl, 0.0, pv)
            cm = cmax2[g, :]
            cg = carg2[g, :]
            resc = ovl & (cg == cstar_f) & (cm > -0.5)
            anyr = _smax(jnp.where(resc, 1.0, 0.0), iota)

            @pl.when(anyr > 0.5)
            def _(g=g, resc=resc, cm=cm, cg=cg):
                def rbody(c, nm_na):
                    nmv, nav = nm_na
                    v = blk[c * NG + g, :]
                    upd = v > nmv
                    return (jnp.where(upd, v, nmv),
                            jnp.where(upd, c.astype(jnp.float32), nav))
                nmax, narg = lax.fori_loop(
                    1, C, rbody,
                    (blk[g, :], jnp.zeros((16,), jnp.float32)))
                cmax2[g, :] = jnp.where(resc, nmax, cm)
                carg2[g, :] = jnp.where(resc, narg, cg)

            killm = (fiota + (base + g * 16).astype(jnp.float32)) == nstar16f
            cmk = cmax2[g, :]
            cmax2[g, :] = jnp.where(killm, -1.0, cmk)
            pdv = preds2[g, :]
            preds2[g, :] = jnp.where(killm, cstar_f, pdv)
        return carry

    lax.fori_loop(0, N, body, jnp.int32(0))
    for g in range(NG):
        predsi[g, :] = preds2[g, :].astype(jnp.int32)
    pltpu.sync_copy(predsi, out_hbm.at[tid])


_sc_call = functools.partial(
    pl.kernel,
    out_type=jax.ShapeDtypeStruct((NT, NG, 16), jnp.int32),
    mesh=plsc.VectorSubcoreMesh(core_axis_name="c", subcore_axis_name="s",
                                num_cores=1, num_subcores=NT),
    compiler_params=pltpu.CompilerParams(use_tc_tiling_on_sc=False),
    scratch_types=[
        pltpu.VMEM((RB * NG, 16), jnp.float32),
        pltpu.VMEM((NG, 16), jnp.float32),
        pltpu.VMEM((NG, 16), jnp.float32),
        pltpu.VMEM((NG, 16), jnp.float32),
        pltpu.VMEM((NG, 16), jnp.int32),
        pltpu.VMEM((16,), jnp.float32),
        pltpu.VMEM((NT, 16), jnp.float32),
        pltpu.VMEM_SHARED((2, NT, 16), jnp.float32),
    ],
)(_sc_greedy)


def kernel(obj_dists1, obj_feats, obj_labels, box_priors, boxes_per_cls, W, b):
    del obj_labels, box_priors
    wt = W.T
    b2 = b[None, :]
    x1t = boxes_per_cls[:, :, 0].T
    y1t = boxes_per_cls[:, :, 1].T
    x2t = boxes_per_cls[:, :, 2].T
    y2t = boxes_per_cls[:, :, 3].T
    d2, pt, areas, cmax, carg = pl.pallas_call(
        _prep_kernel,
        out_shape=(
            jax.ShapeDtypeStruct((N, C), jnp.float32),
            jax.ShapeDtypeStruct((C, N), jnp.float32),
            jax.ShapeDtypeStruct((C, N), jnp.float32),
            jax.ShapeDtypeStruct((1, N), jnp.float32),
            jax.ShapeDtypeStruct((1, N), jnp.float32),
        ),
    )(obj_dists1, obj_feats, wt, b2, x1t, y1t, x2t, y2t)
    pad = NP - N
    stk = jnp.concatenate([
        jnp.pad(pt, ((0, 0), (0, pad)), constant_values=-2.0),
        jnp.pad(x1t, ((0, 0), (0, pad)), constant_values=1.0),
        jnp.pad(y1t, ((0, 0), (0, pad)), constant_values=1.0),
        jnp.pad(x2t, ((0, 0), (0, pad)), constant_values=1.0),
        jnp.pad(y2t, ((0, 0), (0, pad)), constant_values=1.0),
        jnp.pad(areas, ((0, 0), (0, pad)), constant_values=1.0),
        jnp.pad(cmax, ((0, 0), (0, pad)), constant_values=-2.0),
        jnp.pad(carg, ((0, 0), (0, pad)), constant_values=0.0),
    ], axis=0)
    x = stk.reshape(RB, NT, NG, 16).transpose(1, 0, 2, 3) \
        .reshape(NT, RB * NG, 16)
    preds = _sc_call(x)
    return d2, preds.reshape(NP)[:N]


# speedup vs baseline: 1.1656x; 1.1656x over previous
import functools

import jax
import jax.numpy as jnp
from jax import lax
from jax.experimental import pallas as pl
from jax.experimental.pallas import tpu as pltpu
from jax.experimental.pallas import tpu_sc as plsc

N = 1000
C = 51
H = 256
NT = 16
RPT = 64
NG = RPT // 16
NP = NT * RPT
BIG = 2**30

RB = 6 * C + 2

_DNUMS = jax.lax.GatherDimensionNumbers(
    offset_dims=(), collapsed_slice_dims=(0,), start_index_map=(0,))


def _g16(v, idx):
    return jax.lax.gather(v, idx[:, None], _DNUMS, (1,),
                          mode=jax.lax.GatherScatterMode.PROMISE_IN_BOUNDS)


def _bmax(v, iota):
    for sh in (8, 4, 2, 1):
        v = jnp.maximum(v, _g16(v, jnp.bitwise_xor(iota, sh)))
    return v


def _bmin(v, iota):
    for sh in (8, 4, 2, 1):
        v = jnp.minimum(v, _g16(v, jnp.bitwise_xor(iota, sh)))
    return v


def _smax(v, iota):
    for sh in (8, 4, 2):
        v = jnp.maximum(v, _g16(v, jnp.bitwise_xor(iota, sh)))
    return jnp.maximum(v[0], v[1])


def _smin(v, iota):
    for sh in (8, 4, 2):
        v = jnp.minimum(v, _g16(v, jnp.bitwise_xor(iota, sh)))
    return jnp.minimum(v[0], v[1])


def _scal(v, iota):
    return jnp.where(iota == 0, v, jnp.zeros_like(v))[0]


def _prep_kernel(d1_ref, feats_ref, wt_ref, b_ref,
                 x1t_ref, y1t_ref, x2t_ref, y2t_ref,
                 d2_ref, pt_ref, areas_ref, cmax_ref, carg_ref):
    d2 = jnp.dot(feats_ref[...], wt_ref[...],
                 preferred_element_type=jnp.float32)
    d2 = d2 + b_ref[...] + d1_ref[...]
    d2_ref[...] = d2
    d2t = jnp.transpose(d2)
    mx = jnp.max(d2t, axis=0, keepdims=True)
    e = jnp.exp(d2t - mx)
    s = jnp.sum(e, axis=0, keepdims=True)
    pt = e / s
    row = lax.broadcasted_iota(jnp.int32, (C, N), 0)
    pt = jnp.where(row == 0, 0.0, pt)
    pt_ref[...] = pt
    areas_ref[...] = ((x2t_ref[...] - x1t_ref[...]) + 1.0) * \
                     ((y2t_ref[...] - y1t_ref[...]) + 1.0)
    cm = jnp.max(pt, axis=0, keepdims=True)
    cmax_ref[...] = cm
    carg_ref[...] = jnp.min(jnp.where(pt == cm, row, BIG),
                            axis=0, keepdims=True).astype(jnp.float32)


def _sc_greedy(x_hbm, out_hbm, blk, cmax2, carg2, preds2, predsi, rec_v,
               tab_v, dirty, shared):
    tid = lax.axis_index("s")
    base = tid * RPT
    pltpu.sync_copy(x_hbm.at[tid], blk)
    iota = lax.iota(jnp.int32, 16)
    fiota = iota.astype(jnp.float32)

    for g in range(NG):
        cmax2[g, :] = blk[(RB - 2) * NG + g, :]
        carg2[g, :] = blk[(RB - 1) * NG + g, :]
        preds2[g, :] = jnp.zeros((16,), jnp.float32)
    dirty[0, :] = jnp.full((16,), 1.0, jnp.float32)

    def body(it, carry):
        sel = it & 1

        @pl.when(dirty[0, :][0] > 0.5)
        def _():
            cvs = [cmax2[g, :] for g in range(NG)]
            vm = jnp.maximum(jnp.maximum(cvs[0], cvs[1]),
                             jnp.maximum(cvs[2], cvs[3]))
            m16 = _bmax(vm, iota)
            cand = jnp.full((16,), 1024.0, jnp.float32)
            for g in range(NG):
                cand = jnp.minimum(cand, jnp.where(cvs[g] == m16,
                                                   fiota + (g * 16), 1024.0))
            j = _smin(cand, iota).astype(jnp.int32)
            gj = j >> 4
            lj = jnp.full((16,), j & 15, jnp.int32)
            cl16 = _g16(carg2[gj, :], lj)
            cloc = _scal(cl16, iota).astype(jnp.int32)
            x1c = _g16(blk[(C + cloc) * NG + gj, :], lj)
            y1c = _g16(blk[(2 * C + cloc) * NG + gj, :], lj)
            x2c = _g16(blk[(3 * C + cloc) * NG + gj, :], lj)
            y2c = _g16(blk[(4 * C + cloc) * NG + gj, :], lj)
            ac = _g16(blk[(5 * C + cloc) * NG + gj, :], lj)
            nf = (base + j).astype(jnp.float32)
            rec = jnp.where(iota == 0, m16, 0.0)
            rec = jnp.where(iota == 1, nf, rec)
            rec = jnp.where(iota == 2, cl16, rec)
            rec = jnp.where(iota == 3, x1c, rec)
            rec = jnp.where(iota == 4, y1c, rec)
            rec = jnp.where(iota == 5, x2c, rec)
            rec = jnp.where(iota == 6, y2c, rec)
            rec = jnp.where(iota == 7, ac, rec)
            rec_v[...] = rec
        pltpu.sync_copy(rec_v, shared.at[sel, tid])
        plsc.subcore_barrier()
        pltpu.sync_copy(shared.at[sel], tab_v)

        zero16 = jnp.zeros((16,), jnp.int32)
        one16 = jnp.full((16,), 1, jnp.int32)
        vals = jnp.zeros((16,), jnp.float32)
        nvec = jnp.zeros((16,), jnp.float32)
        for t in range(NT):
            rowt = tab_v[t, :]
            vals = jnp.where(iota == t, _g16(rowt, zero16), vals)
            nvec = jnp.where(iota == t, _g16(rowt, one16), nvec)
        gm16 = _bmax(vals, iota)
        nm = vals == gm16
        nstar16 = _bmin(jnp.where(nm, nvec, 1e9), iota)
        tcand = jnp.where(nm & (nvec == nstar16), fiota, 64.0)
        tstar = _smin(tcand, iota).astype(jnp.int32)
        wrec = tab_v[tstar, :]
        nstar16f = _g16(wrec, one16)
        cst16 = _g16(wrec, one16 + 1)
        cstar = _scal(cst16, iota).astype(jnp.int32)
        cstar_f = cst16
        X1 = _g16(wrec, one16 + 2)
        Y1 = _g16(wrec, one16 + 3)
        X2 = _g16(wrec, one16 + 4)
        Y2 = _g16(wrec, one16 + 5)
        A = _g16(wrec, one16 + 6)

        anyrs = []
        for g in range(NG):
            pv = blk[cstar * NG + g, :]
            x1v = blk[(C + cstar) * NG + g, :]
            y1v = blk[(2 * C + cstar) * NG + g, :]
            x2v = blk[(3 * C + cstar) * NG + g, :]
            y2v = blk[(4 * C + cstar) * NG + g, :]
            av = blk[(5 * C + cstar) * NG + g, :]
            iw = jnp.maximum(jnp.minimum(X2, x2v) - jnp.maximum(X1, x1v)
                             + 1.0, 0.0)
            ih = jnp.maximum(jnp.minimum(Y2, y2v) - jnp.maximum(Y1, y1v)
                             + 1.0, 0.0)
            inters = iw * ih
            union = (-inters + av) + A
            ovl = (inters / union) >= 0.5
            blk[cstar * NG + g, :] = jnp.where(ovl, 0.0, pv)
            cm = cmax2[g, :]
            cg = carg2[g, :]
            resc = ovl & (cg == cstar_f) & (cm > -0.5)
            anyr = _smax(jnp.where(resc, 1.0, 0.0), iota)
            anyrs.append(anyr)

            @pl.when(anyr > 0.5)
            def _(g=g, resc=resc, cm=cm, cg=cg):
                nmax = blk[g, :]
                narg = jnp.zeros((16,), jnp.float32)
                for c in range(1, C):
                    v = blk[c * NG + g, :]
                    upd = v > nmax
                    nmax = jnp.where(upd, v, nmax)
                    narg = jnp.where(upd, float(c), narg)
                cmax2[g, :] = jnp.where(resc, nmax, cm)
                carg2[g, :] = jnp.where(resc, narg, cg)

            killm = (fiota + (base + g * 16).astype(jnp.float32)) == nstar16f
            cmk = cmax2[g, :]
            cmax2[g, :] = jnp.where(killm, -1.0, cmk)
            pdv = preds2[g, :]
            preds2[g, :] = jnp.where(killm, cstar_f, pdv)

        ns = _scal(nstar16f, iota)
        basef = base.astype(jnp.float32)
        kill_mine = (ns >= basef) & (ns < basef + float(RPT))
        dmax = jnp.maximum(jnp.maximum(anyrs[0], anyrs[1]),
                           jnp.maximum(anyrs[2], anyrs[3]))
        dflag = jnp.where(kill_mine, 1.0, dmax)
        dirty[0, :] = jnp.full((16,), dflag, jnp.float32)
        return carry

    lax.fori_loop(0, N, body, jnp.int32(0))
    for g in range(NG):
        predsi[g, :] = preds2[g, :].astype(jnp.int32)
    pltpu.sync_copy(predsi, out_hbm.at[tid])


_sc_call = functools.partial(
    pl.kernel,
    out_type=jax.ShapeDtypeStruct((NT, NG, 16), jnp.int32),
    mesh=plsc.VectorSubcoreMesh(core_axis_name="c", subcore_axis_name="s",
                                num_cores=1, num_subcores=NT),
    compiler_params=pltpu.CompilerParams(use_tc_tiling_on_sc=False),
    scratch_types=[
        pltpu.VMEM((RB * NG, 16), jnp.float32),
        pltpu.VMEM((NG, 16), jnp.float32),
        pltpu.VMEM((NG, 16), jnp.float32),
        pltpu.VMEM((NG, 16), jnp.float32),
        pltpu.VMEM((NG, 16), jnp.int32),
        pltpu.VMEM((16,), jnp.float32),
        pltpu.VMEM((NT, 16), jnp.float32),
        pltpu.VMEM((1, 16), jnp.float32),
        pltpu.VMEM_SHARED((2, NT, 16), jnp.float32),
    ],
)(_sc_greedy)


def kernel(obj_dists1, obj_feats, obj_labels, box_priors, boxes_per_cls, W, b):
    del obj_labels, box_priors
    wt = W.T
    b2 = b[None, :]
    x1t = boxes_per_cls[:, :, 0].T
    y1t = boxes_per_cls[:, :, 1].T
    x2t = boxes_per_cls[:, :, 2].T
    y2t = boxes_per_cls[:, :, 3].T
    d2, pt, areas, cmax, carg = pl.pallas_call(
        _prep_kernel,
        out_shape=(
            jax.ShapeDtypeStruct((N, C), jnp.float32),
            jax.ShapeDtypeStruct((C, N), jnp.float32),
            jax.ShapeDtypeStruct((C, N), jnp.float32),
            jax.ShapeDtypeStruct((1, N), jnp.float32),
            jax.ShapeDtypeStruct((1, N), jnp.float32),
        ),
    )(obj_dists1, obj_feats, wt, b2, x1t, y1t, x2t, y2t)
    pad = NP - N
    stk = jnp.concatenate([
        jnp.pad(pt, ((0, 0), (0, pad)), constant_values=-2.0),
        jnp.pad(x1t, ((0, 0), (0, pad)), constant_values=1.0),
        jnp.pad(y1t, ((0, 0), (0, pad)), constant_values=1.0),
        jnp.pad(x2t, ((0, 0), (0, pad)), constant_values=1.0),
        jnp.pad(y2t, ((0, 0), (0, pad)), constant_values=1.0),
        jnp.pad(areas, ((0, 0), (0, pad)), constant_values=1.0),
        jnp.pad(cmax, ((0, 0), (0, pad)), constant_values=-2.0),
        jnp.pad(carg, ((0, 0), (0, pad)), constant_values=0.0),
    ], axis=0)
    x = stk.reshape(RB, NT, NG, 16).transpose(1, 0, 2, 3) \
        .reshape(NT, RB * NG, 16)
    preds = _sc_call(x)
    return d2, preds.reshape(NP)[:N]


# speedup vs baseline: 1.1731x; 1.0064x over previous
import functools

import jax
import jax.numpy as jnp
from jax import lax
from jax.experimental import pallas as pl
from jax.experimental.pallas import tpu as pltpu
from jax.experimental.pallas import tpu_sc as plsc

N = 1000
C = 51
H = 256
NT = 16
RPT = 64
NG = RPT // 16
NP = NT * RPT
BIG = 2**30

RB = 6 * C + 2

_DNUMS = jax.lax.GatherDimensionNumbers(
    offset_dims=(), collapsed_slice_dims=(0,), start_index_map=(0,))


def _g16(v, idx):
    return jax.lax.gather(v, idx[:, None], _DNUMS, (1,),
                          mode=jax.lax.GatherScatterMode.PROMISE_IN_BOUNDS)


def _bmax(v, iota):
    for sh in (8, 4, 2, 1):
        v = jnp.maximum(v, _g16(v, jnp.bitwise_xor(iota, sh)))
    return v


def _bmin(v, iota):
    for sh in (8, 4, 2, 1):
        v = jnp.minimum(v, _g16(v, jnp.bitwise_xor(iota, sh)))
    return v


def _smax(v, iota):
    for sh in (8, 4, 2):
        v = jnp.maximum(v, _g16(v, jnp.bitwise_xor(iota, sh)))
    return jnp.maximum(v[0], v[1])


def _smin(v, iota):
    for sh in (8, 4, 2):
        v = jnp.minimum(v, _g16(v, jnp.bitwise_xor(iota, sh)))
    return jnp.minimum(v[0], v[1])


def _scal(v, iota):
    return jnp.where(iota == 0, v, jnp.zeros_like(v))[0]


def _prep_kernel(d1_ref, feats_ref, wt_ref, b_ref,
                 x1t_ref, y1t_ref, x2t_ref, y2t_ref,
                 d2_ref, pt_ref, areas_ref, cmax_ref, carg_ref):
    d2 = jnp.dot(feats_ref[...], wt_ref[...],
                 preferred_element_type=jnp.float32)
    d2 = d2 + b_ref[...] + d1_ref[...]
    d2_ref[...] = d2
    d2t = jnp.transpose(d2)
    mx = jnp.max(d2t, axis=0, keepdims=True)
    e = jnp.exp(d2t - mx)
    s = jnp.sum(e, axis=0, keepdims=True)
    pt = e / s
    row = lax.broadcasted_iota(jnp.int32, (C, N), 0)
    pt = jnp.where(row == 0, 0.0, pt)
    pt_ref[...] = pt
    areas_ref[...] = ((x2t_ref[...] - x1t_ref[...]) + 1.0) * \
                     ((y2t_ref[...] - y1t_ref[...]) + 1.0)
    cm = jnp.max(pt, axis=0, keepdims=True)
    cmax_ref[...] = cm
    carg_ref[...] = jnp.min(jnp.where(pt == cm, row, BIG),
                            axis=0, keepdims=True).astype(jnp.float32)


def _sc_greedy(x_hbm, out_hbm, blk, cmax2, carg2, preds2, predsi, rec_v,
               tab_v, dirty, shared):
    tid = lax.axis_index("s")
    base = tid * RPT
    pltpu.sync_copy(x_hbm.at[tid], blk)
    iota = lax.iota(jnp.int32, 16)
    fiota = iota.astype(jnp.float32)

    for g in range(NG):
        cmax2[g, :] = blk[(RB - 2) * NG + g, :]
        carg2[g, :] = blk[(RB - 1) * NG + g, :]
        preds2[g, :] = jnp.zeros((16,), jnp.float32)
    dirty[0, :] = jnp.full((16,), 1.0, jnp.float32)

    def body(it, carry):
        sel = it & 1

        @pl.when(dirty[0, :][0] > 0.5)
        def _():
            cvs = [cmax2[g, :] for g in range(NG)]
            vm = jnp.maximum(jnp.maximum(cvs[0], cvs[1]),
                             jnp.maximum(cvs[2], cvs[3]))
            m16 = _bmax(vm, iota)
            cand = jnp.full((16,), 1024.0, jnp.float32)
            for g in range(NG):
                cand = jnp.minimum(cand, jnp.where(cvs[g] == m16,
                                                   fiota + (g * 16), 1024.0))
            j = _smin(cand, iota).astype(jnp.int32)
            gj = j >> 4
            lj = jnp.full((16,), j & 15, jnp.int32)
            cl16 = _g16(carg2[gj, :], lj)
            cloc = _scal(cl16, iota).astype(jnp.int32)
            x1c = _g16(blk[(C + cloc) * NG + gj, :], lj)
            y1c = _g16(blk[(2 * C + cloc) * NG + gj, :], lj)
            x2c = _g16(blk[(3 * C + cloc) * NG + gj, :], lj)
            y2c = _g16(blk[(4 * C + cloc) * NG + gj, :], lj)
            ac = _g16(blk[(5 * C + cloc) * NG + gj, :], lj)
            nf = (base + j).astype(jnp.float32)
            rec = jnp.where(iota == 0, m16, 0.0)
            rec = jnp.where(iota == 1, nf, rec)
            rec = jnp.where(iota == 2, cl16, rec)
            rec = jnp.where(iota == 3, x1c, rec)
            rec = jnp.where(iota == 4, y1c, rec)
            rec = jnp.where(iota == 5, x2c, rec)
            rec = jnp.where(iota == 6, y2c, rec)
            rec = jnp.where(iota == 7, ac, rec)
            rec_v[...] = rec
        pltpu.sync_copy(rec_v.at[pl.ds(0, 8)],
                        shared.at[sel, tid >> 1, pl.ds((tid & 1) * 8, 8)])
        plsc.subcore_barrier()
        pltpu.sync_copy(shared.at[sel], tab_v)

        zero16 = jnp.zeros((16,), jnp.int32)
        one16 = jnp.full((16,), 1, jnp.int32)
        eight16 = jnp.full((16,), 8, jnp.int32)
        vals = jnp.zeros((16,), jnp.float32)
        nvec = jnp.zeros((16,), jnp.float32)
        for r in range(NT // 2):
            rowt = tab_v[r, :]
            vals = jnp.where(iota == 2 * r, _g16(rowt, zero16), vals)
            vals = jnp.where(iota == 2 * r + 1, _g16(rowt, eight16), vals)
            nvec = jnp.where(iota == 2 * r, _g16(rowt, one16), nvec)
            nvec = jnp.where(iota == 2 * r + 1, _g16(rowt, eight16 + 1), nvec)
        gm16 = _bmax(vals, iota)
        nm = vals == gm16
        nstar16 = _bmin(jnp.where(nm, nvec, 1e9), iota)
        tcand = jnp.where(nm & (nvec == nstar16), fiota, 64.0)
        tstar = _smin(tcand, iota).astype(jnp.int32)
        wrec = tab_v[tstar >> 1, :]
        off16 = jnp.full((16,), (tstar & 1) * 8, jnp.int32)
        nstar16f = _g16(wrec, off16 + 1)
        cst16 = _g16(wrec, off16 + 2)
        cstar = _scal(cst16, iota).astype(jnp.int32)
        cstar_f = cst16
        X1 = _g16(wrec, off16 + 3)
        Y1 = _g16(wrec, off16 + 4)
        X2 = _g16(wrec, off16 + 5)
        Y2 = _g16(wrec, off16 + 6)
        A = _g16(wrec, off16 + 7)

        anyrs = []
        for g in range(NG):
            pv = blk[cstar * NG + g, :]
            x1v = blk[(C + cstar) * NG + g, :]
            y1v = blk[(2 * C + cstar) * NG + g, :]
            x2v = blk[(3 * C + cstar) * NG + g, :]
            y2v = blk[(4 * C + cstar) * NG + g, :]
            av = blk[(5 * C + cstar) * NG + g, :]
            iw = jnp.maximum(jnp.minimum(X2, x2v) - jnp.maximum(X1, x1v)
                             + 1.0, 0.0)
            ih = jnp.maximum(jnp.minimum(Y2, y2v) - jnp.maximum(Y1, y1v)
                             + 1.0, 0.0)
            inters = iw * ih
            union = (-inters + av) + A
            ovl = (inters / union) >= 0.5
            blk[cstar * NG + g, :] = jnp.where(ovl, 0.0, pv)
            cm = cmax2[g, :]
            cg = carg2[g, :]
            resc = ovl & (cg == cstar_f) & (cm > -0.5)
            anyr = _smax(jnp.where(resc, 1.0, 0.0), iota)
            anyrs.append(anyr)

            @pl.when(anyr > 0.5)
            def _(g=g, resc=resc, cm=cm, cg=cg):
                nmax = blk[g, :]
                narg = jnp.zeros((16,), jnp.float32)
                for c in range(1, C):
                    v = blk[c * NG + g, :]
                    upd = v > nmax
                    nmax = jnp.where(upd, v, nmax)
                    narg = jnp.where(upd, float(c), narg)
                cmax2[g, :] = jnp.where(resc, nmax, cm)
                carg2[g, :] = jnp.where(resc, narg, cg)

            killm = (fiota + (base + g * 16).astype(jnp.float32)) == nstar16f
            cmk = cmax2[g, :]
            cmax2[g, :] = jnp.where(killm, -1.0, cmk)
            pdv = preds2[g, :]
            preds2[g, :] = jnp.where(killm, cstar_f, pdv)

        ns = _scal(nstar16f, iota)
        basef = base.astype(jnp.float32)
        kill_mine = (ns >= basef) & (ns < basef + float(RPT))
        dmax = jnp.maximum(jnp.maximum(anyrs[0], anyrs[1]),
                           jnp.maximum(anyrs[2], anyrs[3]))
        dflag = jnp.where(kill_mine, 1.0, dmax)
        dirty[0, :] = jnp.full((16,), dflag, jnp.float32)
        return carry

    lax.fori_loop(0, N, body, jnp.int32(0))
    for g in range(NG):
        predsi[g, :] = preds2[g, :].astype(jnp.int32)
    pltpu.sync_copy(predsi, out_hbm.at[tid])


_sc_call = functools.partial(
    pl.kernel,
    out_type=jax.ShapeDtypeStruct((NT, NG, 16), jnp.int32),
    mesh=plsc.VectorSubcoreMesh(core_axis_name="c", subcore_axis_name="s",
                                num_cores=1, num_subcores=NT),
    compiler_params=pltpu.CompilerParams(use_tc_tiling_on_sc=False),
    scratch_types=[
        pltpu.VMEM((RB * NG, 16), jnp.float32),
        pltpu.VMEM((NG, 16), jnp.float32),
        pltpu.VMEM((NG, 16), jnp.float32),
        pltpu.VMEM((NG, 16), jnp.float32),
        pltpu.VMEM((NG, 16), jnp.int32),
        pltpu.VMEM((16,), jnp.float32),
        pltpu.VMEM((NT // 2, 16), jnp.float32),
        pltpu.VMEM((1, 16), jnp.float32),
        pltpu.VMEM_SHARED((2, NT // 2, 16), jnp.float32),
    ],
)(_sc_greedy)


def kernel(obj_dists1, obj_feats, obj_labels, box_priors, boxes_per_cls, W, b):
    del obj_labels, box_priors
    wt = W.T
    b2 = b[None, :]
    x1t = boxes_per_cls[:, :, 0].T
    y1t = boxes_per_cls[:, :, 1].T
    x2t = boxes_per_cls[:, :, 2].T
    y2t = boxes_per_cls[:, :, 3].T
    d2, pt, areas, cmax, carg = pl.pallas_call(
        _prep_kernel,
        out_shape=(
            jax.ShapeDtypeStruct((N, C), jnp.float32),
            jax.ShapeDtypeStruct((C, N), jnp.float32),
            jax.ShapeDtypeStruct((C, N), jnp.float32),
            jax.ShapeDtypeStruct((1, N), jnp.float32),
            jax.ShapeDtypeStruct((1, N), jnp.float32),
        ),
    )(obj_dists1, obj_feats, wt, b2, x1t, y1t, x2t, y2t)
    pad = NP - N
    stk = jnp.concatenate([
        jnp.pad(pt, ((0, 0), (0, pad)), constant_values=-2.0),
        jnp.pad(x1t, ((0, 0), (0, pad)), constant_values=1.0),
        jnp.pad(y1t, ((0, 0), (0, pad)), constant_values=1.0),
        jnp.pad(x2t, ((0, 0), (0, pad)), constant_values=1.0),
        jnp.pad(y2t, ((0, 0), (0, pad)), constant_values=1.0),
        jnp.pad(areas, ((0, 0), (0, pad)), constant_values=1.0),
        jnp.pad(cmax, ((0, 0), (0, pad)), constant_values=-2.0),
        jnp.pad(carg, ((0, 0), (0, pad)), constant_values=0.0),
    ], axis=0)
    x = stk.reshape(RB, NT, NG, 16).transpose(1, 0, 2, 3) \
        .reshape(NT, RB * NG, 16)
    preds = _sc_call(x)
    return d2, preds.reshape(NP)[:N]


# speedup vs baseline: 1.2946x; 1.1036x over previous
import functools

import jax
import jax.numpy as jnp
from jax import lax
from jax.experimental import pallas as pl
from jax.experimental.pallas import tpu as pltpu
from jax.experimental.pallas import tpu_sc as plsc

N = 1000
C = 51
H = 256
NT = 16
RPT = 64
NG = RPT // 16
NP = NT * RPT
BIG = 2**30

RB = 6 * C + 2

_DNUMS = jax.lax.GatherDimensionNumbers(
    offset_dims=(), collapsed_slice_dims=(0,), start_index_map=(0,))


def _g16(v, idx):
    return jax.lax.gather(v, idx[:, None], _DNUMS, (1,),
                          mode=jax.lax.GatherScatterMode.PROMISE_IN_BOUNDS)


def _bmax(v, iota):
    for sh in (8, 4, 2, 1):
        v = jnp.maximum(v, _g16(v, jnp.bitwise_xor(iota, sh)))
    return v


def _bmin(v, iota):
    for sh in (8, 4, 2, 1):
        v = jnp.minimum(v, _g16(v, jnp.bitwise_xor(iota, sh)))
    return v


def _smax(v, iota):
    for sh in (8, 4, 2):
        v = jnp.maximum(v, _g16(v, jnp.bitwise_xor(iota, sh)))
    return jnp.maximum(v[0], v[1])


def _smin(v, iota):
    for sh in (8, 4, 2):
        v = jnp.minimum(v, _g16(v, jnp.bitwise_xor(iota, sh)))
    return jnp.minimum(v[0], v[1])


def _scal(v, iota):
    return jnp.where(iota == 0, v, jnp.zeros_like(v))[0]


def _prep_kernel(d1_ref, feats_ref, wt_ref, b_ref,
                 x1t_ref, y1t_ref, x2t_ref, y2t_ref,
                 d2_ref, pt_ref, areas_ref, cmax_ref, carg_ref):
    d2 = jnp.dot(feats_ref[...], wt_ref[...],
                 preferred_element_type=jnp.float32)
    d2 = d2 + b_ref[...] + d1_ref[...]
    d2_ref[...] = d2
    d2t = jnp.transpose(d2)
    mx = jnp.max(d2t, axis=0, keepdims=True)
    e = jnp.exp(d2t - mx)
    s = jnp.sum(e, axis=0, keepdims=True)
    pt = e / s
    row = lax.broadcasted_iota(jnp.int32, (C, N), 0)
    pt = jnp.where(row == 0, 0.0, pt)
    pt_ref[...] = pt
    areas_ref[...] = ((x2t_ref[...] - x1t_ref[...]) + 1.0) * \
                     ((y2t_ref[...] - y1t_ref[...]) + 1.0)
    cm = jnp.max(pt, axis=0, keepdims=True)
    cmax_ref[...] = cm
    carg_ref[...] = jnp.min(jnp.where(pt == cm, row, BIG),
                            axis=0, keepdims=True).astype(jnp.float32)


def _sc_greedy(x_hbm, out_hbm, blk, cmax2, carg2, preds2, predsi, rec_v,
               tab_v, dirty, shared):
    tid = lax.axis_index("s")
    base = tid * RPT
    pltpu.sync_copy(x_hbm.at[tid], blk)
    iota = lax.iota(jnp.int32, 16)
    fiota = iota.astype(jnp.float32)

    for g in range(NG):
        cmax2[g, :] = blk[(RB - 2) * NG + g, :]
        carg2[g, :] = blk[(RB - 1) * NG + g, :]
        preds2[g, :] = jnp.zeros((16,), jnp.float32)
    dirty[0, :] = jnp.full((16,), 1.0, jnp.float32)

    def body(it, carry):
        sel = it & 1

        @pl.when(dirty[0, :][0] > 0.5)
        def _():
            cvs = [cmax2[g, :] for g in range(NG)]
            vm = jnp.maximum(jnp.maximum(cvs[0], cvs[1]),
                             jnp.maximum(cvs[2], cvs[3]))
            m16 = _bmax(vm, iota)
            cand = jnp.full((16,), 1024.0, jnp.float32)
            for g in range(NG):
                cand = jnp.minimum(cand, jnp.where(cvs[g] == m16,
                                                   fiota + (g * 16), 1024.0))
            j = _smin(cand, iota).astype(jnp.int32)
            gj = j >> 4
            lj = jnp.full((16,), j & 15, jnp.int32)
            cl16 = _g16(carg2[gj, :], lj)
            cloc = _scal(cl16, iota).astype(jnp.int32)
            x1c = _g16(blk[(C + cloc) * NG + gj, :], lj)
            y1c = _g16(blk[(2 * C + cloc) * NG + gj, :], lj)
            x2c = _g16(blk[(3 * C + cloc) * NG + gj, :], lj)
            y2c = _g16(blk[(4 * C + cloc) * NG + gj, :], lj)
            ac = _g16(blk[(5 * C + cloc) * NG + gj, :], lj)
            nf = (base + j).astype(jnp.float32)
            rec = jnp.where(iota == 0, m16, 0.0)
            rec = jnp.where(iota == 1, nf, rec)
            rec = jnp.where(iota == 2, cl16, rec)
            rec = jnp.where(iota == 3, x1c, rec)
            rec = jnp.where(iota == 4, y1c, rec)
            rec = jnp.where(iota == 5, x2c, rec)
            rec = jnp.where(iota == 6, y2c, rec)
            rec = jnp.where(iota == 7, ac, rec)
            rec_v[...] = rec
        pltpu.sync_copy(rec_v.at[pl.ds(0, 8)],
                        shared.at[sel, tid >> 1, pl.ds((tid & 1) * 8, 8)])
        plsc.subcore_barrier()
        pltpu.sync_copy(shared.at[sel], tab_v)

        zero16 = jnp.zeros((16,), jnp.int32)
        one16 = jnp.full((16,), 1, jnp.int32)
        eight16 = jnp.full((16,), 8, jnp.int32)
        vals = jnp.zeros((16,), jnp.float32)
        nvec = jnp.zeros((16,), jnp.float32)
        for r in range(NT // 2):
            rowt = tab_v[r, :]
            vals = jnp.where(iota == 2 * r, _g16(rowt, zero16), vals)
            vals = jnp.where(iota == 2 * r + 1, _g16(rowt, eight16), vals)
            nvec = jnp.where(iota == 2 * r, _g16(rowt, one16), nvec)
            nvec = jnp.where(iota == 2 * r + 1, _g16(rowt, eight16 + 1), nvec)
        gm16 = _bmax(vals, iota)
        nm = vals == gm16
        nstar16 = _bmin(jnp.where(nm, nvec, 1e9), iota)
        tcand = jnp.where(nm & (nvec == nstar16), fiota, 64.0)
        tstar = _smin(tcand, iota).astype(jnp.int32)
        wrec = tab_v[tstar >> 1, :]
        off16 = jnp.full((16,), (tstar & 1) * 8, jnp.int32)
        nstar16f = _g16(wrec, off16 + 1)
        cst16 = _g16(wrec, off16 + 2)
        cstar = _scal(cst16, iota).astype(jnp.int32)
        cstar_f = cst16
        X1 = _g16(wrec, off16 + 3)
        Y1 = _g16(wrec, off16 + 4)
        X2 = _g16(wrec, off16 + 5)
        Y2 = _g16(wrec, off16 + 6)
        A = _g16(wrec, off16 + 7)

        anyrs = []
        for g in range(NG):
            pv = blk[cstar * NG + g, :]
            x1v = blk[(C + cstar) * NG + g, :]
            y1v = blk[(2 * C + cstar) * NG + g, :]
            x2v = blk[(3 * C + cstar) * NG + g, :]
            y2v = blk[(4 * C + cstar) * NG + g, :]
            av = blk[(5 * C + cstar) * NG + g, :]
            iw = jnp.maximum(jnp.minimum(X2, x2v) - jnp.maximum(X1, x1v)
                             + 1.0, 0.0)
            ih = jnp.maximum(jnp.minimum(Y2, y2v) - jnp.maximum(Y1, y1v)
                             + 1.0, 0.0)
            inters = iw * ih
            union = (-inters + av) + A
            ovl = (inters / union) >= 0.5
            blk[cstar * NG + g, :] = jnp.where(ovl, 0.0, pv)
            cm = cmax2[g, :]
            cg = carg2[g, :]
            gidf = fiota + (base + g * 16).astype(jnp.float32)
            killm = gidf == nstar16f
            resc = ovl & (cg == cstar_f) & (cm > -0.5) & (gidf != nstar16f)
            anyr = _smax(jnp.where(resc, 1.0, 0.0), iota)
            anyrs.append(anyr)

            @pl.when(anyr > 0.5)
            def _(g=g, resc=resc, cm=cm, cg=cg):
                nmax = blk[g, :]
                narg = jnp.zeros((16,), jnp.float32)
                for c in range(1, C):
                    v = blk[c * NG + g, :]
                    upd = v > nmax
                    nmax = jnp.where(upd, v, nmax)
                    narg = jnp.where(upd, float(c), narg)
                cmax2[g, :] = jnp.where(resc, nmax, cm)
                carg2[g, :] = jnp.where(resc, narg, cg)

            cmk = cmax2[g, :]
            cmax2[g, :] = jnp.where(killm, -1.0, cmk)
            pdv = preds2[g, :]
            preds2[g, :] = jnp.where(killm, cstar_f, pdv)

        ns = _scal(nstar16f, iota)
        basef = base.astype(jnp.float32)
        kill_mine = (ns >= basef) & (ns < basef + float(RPT))
        dmax = jnp.maximum(jnp.maximum(anyrs[0], anyrs[1]),
                           jnp.maximum(anyrs[2], anyrs[3]))
        dflag = jnp.where(kill_mine, 1.0, dmax)
        dirty[0, :] = jnp.full((16,), dflag, jnp.float32)
        return carry

    lax.fori_loop(0, N, body, jnp.int32(0))
    for g in range(NG):
        predsi[g, :] = preds2[g, :].astype(jnp.int32)
    pltpu.sync_copy(predsi, out_hbm.at[tid])


_sc_call = functools.partial(
    pl.kernel,
    out_type=jax.ShapeDtypeStruct((NT, NG, 16), jnp.int32),
    mesh=plsc.VectorSubcoreMesh(core_axis_name="c", subcore_axis_name="s",
                                num_cores=1, num_subcores=NT),
    compiler_params=pltpu.CompilerParams(use_tc_tiling_on_sc=False),
    scratch_types=[
        pltpu.VMEM((RB * NG, 16), jnp.float32),
        pltpu.VMEM((NG, 16), jnp.float32),
        pltpu.VMEM((NG, 16), jnp.float32),
        pltpu.VMEM((NG, 16), jnp.float32),
        pltpu.VMEM((NG, 16), jnp.int32),
        pltpu.VMEM((16,), jnp.float32),
        pltpu.VMEM((NT // 2, 16), jnp.float32),
        pltpu.VMEM((1, 16), jnp.float32),
        pltpu.VMEM_SHARED((2, NT // 2, 16), jnp.float32),
    ],
)(_sc_greedy)


def kernel(obj_dists1, obj_feats, obj_labels, box_priors, boxes_per_cls, W, b):
    del obj_labels, box_priors
    wt = W.T
    b2 = b[None, :]
    x1t = boxes_per_cls[:, :, 0].T
    y1t = boxes_per_cls[:, :, 1].T
    x2t = boxes_per_cls[:, :, 2].T
    y2t = boxes_per_cls[:, :, 3].T
    d2, pt, areas, cmax, carg = pl.pallas_call(
        _prep_kernel,
        out_shape=(
            jax.ShapeDtypeStruct((N, C), jnp.float32),
            jax.ShapeDtypeStruct((C, N), jnp.float32),
            jax.ShapeDtypeStruct((C, N), jnp.float32),
            jax.ShapeDtypeStruct((1, N), jnp.float32),
            jax.ShapeDtypeStruct((1, N), jnp.float32),
        ),
    )(obj_dists1, obj_feats, wt, b2, x1t, y1t, x2t, y2t)
    pad = NP - N
    stk = jnp.concatenate([
        jnp.pad(pt, ((0, 0), (0, pad)), constant_values=-2.0),
        jnp.pad(x1t, ((0, 0), (0, pad)), constant_values=1.0),
        jnp.pad(y1t, ((0, 0), (0, pad)), constant_values=1.0),
        jnp.pad(x2t, ((0, 0), (0, pad)), constant_values=1.0),
        jnp.pad(y2t, ((0, 0), (0, pad)), constant_values=1.0),
        jnp.pad(areas, ((0, 0), (0, pad)), constant_values=1.0),
        jnp.pad(cmax, ((0, 0), (0, pad)), constant_values=-2.0),
        jnp.pad(carg, ((0, 0), (0, pad)), constant_values=0.0),
    ], axis=0)
    x = stk.reshape(RB, NT, NG, 16).transpose(1, 0, 2, 3) \
        .reshape(NT, RB * NG, 16)
    preds = _sc_call(x)
    return d2, preds.reshape(NP)[:N]


# speedup vs baseline: 1.3247x; 1.0232x over previous
import functools

import jax
import jax.numpy as jnp
from jax import lax
from jax.experimental import pallas as pl
from jax.experimental.pallas import tpu as pltpu
from jax.experimental.pallas import tpu_sc as plsc

N = 1000
C = 51
H = 256
NT = 16
RPT = 64
NG = RPT // 16
NP = NT * RPT
BIG = 2**30

RB = 6 * C + 2

_DNUMS = jax.lax.GatherDimensionNumbers(
    offset_dims=(), collapsed_slice_dims=(0,), start_index_map=(0,))


def _g16(v, idx):
    return jax.lax.gather(v, idx[:, None], _DNUMS, (1,),
                          mode=jax.lax.GatherScatterMode.PROMISE_IN_BOUNDS)


def _bmax(v, iota):
    for sh in (8, 4, 2, 1):
        v = jnp.maximum(v, _g16(v, jnp.bitwise_xor(iota, sh)))
    return v


def _bmin(v, iota):
    for sh in (8, 4, 2, 1):
        v = jnp.minimum(v, _g16(v, jnp.bitwise_xor(iota, sh)))
    return v


def _smax(v, iota):
    for sh in (8, 4, 2):
        v = jnp.maximum(v, _g16(v, jnp.bitwise_xor(iota, sh)))
    return jnp.maximum(v[0], v[1])


def _smin(v, iota):
    for sh in (8, 4, 2):
        v = jnp.minimum(v, _g16(v, jnp.bitwise_xor(iota, sh)))
    return jnp.minimum(v[0], v[1])


def _scal(v, iota):
    return jnp.where(iota == 0, v, jnp.zeros_like(v))[0]


def _prep_kernel(d1_ref, feats_ref, w_ref, b_ref,
                 x1_ref, y1_ref, x2_ref, y2_ref,
                 d2_ref, stk_ref):
    d2 = lax.dot_general(feats_ref[...], w_ref[...],
                         (((1,), (1,)), ((), ())),
                         preferred_element_type=jnp.float32)
    d2 = d2 + b_ref[...] + d1_ref[...]
    d2_ref[...] = d2
    d2t = jnp.transpose(d2)
    mx = jnp.max(d2t, axis=0, keepdims=True)
    e = jnp.exp(d2t - mx)
    s = jnp.sum(e, axis=0, keepdims=True)
    pt = e / s
    row = lax.broadcasted_iota(jnp.int32, (C, N), 0)
    pt = jnp.where(row == 0, 0.0, pt)
    x1t = jnp.transpose(x1_ref[...])
    y1t = jnp.transpose(y1_ref[...])
    x2t = jnp.transpose(x2_ref[...])
    y2t = jnp.transpose(y2_ref[...])
    areas = ((x2t - x1t) + 1.0) * ((y2t - y1t) + 1.0)
    cm = jnp.max(pt, axis=0, keepdims=True)
    cg = jnp.min(jnp.where(pt == cm, row, BIG),
                 axis=0, keepdims=True).astype(jnp.float32)
    pad = NP - N
    ptp = jnp.concatenate([pt, jnp.full((C, pad), -2.0, jnp.float32)], axis=1)
    onep = jnp.full((C, pad), 1.0, jnp.float32)
    x1p = jnp.concatenate([x1t, onep], axis=1)
    y1p = jnp.concatenate([y1t, onep], axis=1)
    x2p = jnp.concatenate([x2t, onep], axis=1)
    y2p = jnp.concatenate([y2t, onep], axis=1)
    arp = jnp.concatenate([areas, onep], axis=1)
    cmp_ = jnp.concatenate([cm, jnp.full((1, pad), -2.0, jnp.float32)],
                           axis=1)
    cgp = jnp.concatenate([cg, jnp.zeros((1, pad), jnp.float32)], axis=1)
    for t in range(NT):
        lo, hi = t * RPT, (t + 1) * RPT
        stk_ref[t, 0:C, :] = ptp[:, lo:hi]
        stk_ref[t, C:2 * C, :] = x1p[:, lo:hi]
        stk_ref[t, 2 * C:3 * C, :] = y1p[:, lo:hi]
        stk_ref[t, 3 * C:4 * C, :] = x2p[:, lo:hi]
        stk_ref[t, 4 * C:5 * C, :] = y2p[:, lo:hi]
        stk_ref[t, 5 * C:6 * C, :] = arp[:, lo:hi]
        stk_ref[t, 6 * C:6 * C + 1, :] = cmp_[:, lo:hi]
        stk_ref[t, 6 * C + 1:RB, :] = cgp[:, lo:hi]


def _sc_greedy(x_hbm, out_hbm, blk, cmax2, carg2, preds2, predsi, rec_v,
               tab_v, dirty, shared):
    tid = lax.axis_index("s")
    base = tid * RPT
    pltpu.sync_copy(x_hbm.at[tid], blk)
    iota = lax.iota(jnp.int32, 16)
    fiota = iota.astype(jnp.float32)

    for g in range(NG):
        cmax2[g, :] = blk[(RB - 2) * NG + g, :]
        carg2[g, :] = blk[(RB - 1) * NG + g, :]
        preds2[g, :] = jnp.zeros((16,), jnp.float32)
    dirty[0, :] = jnp.full((16,), 1.0, jnp.float32)

    def body(it, carry):
        sel = it & 1

        @pl.when(dirty[0, :][0] > 0.5)
        def _():
            cvs = [cmax2[g, :] for g in range(NG)]
            vm = jnp.maximum(jnp.maximum(cvs[0], cvs[1]),
                             jnp.maximum(cvs[2], cvs[3]))
            m16 = _bmax(vm, iota)
            cand = jnp.full((16,), 1024.0, jnp.float32)
            for g in range(NG):
                cand = jnp.minimum(cand, jnp.where(cvs[g] == m16,
                                                   fiota + (g * 16), 1024.0))
            j = _smin(cand, iota).astype(jnp.int32)
            gj = j >> 4
            lj = jnp.full((16,), j & 15, jnp.int32)
            cl16 = _g16(carg2[gj, :], lj)
            cloc = _scal(cl16, iota).astype(jnp.int32)
            x1c = _g16(blk[(C + cloc) * NG + gj, :], lj)
            y1c = _g16(blk[(2 * C + cloc) * NG + gj, :], lj)
            x2c = _g16(blk[(3 * C + cloc) * NG + gj, :], lj)
            y2c = _g16(blk[(4 * C + cloc) * NG + gj, :], lj)
            ac = _g16(blk[(5 * C + cloc) * NG + gj, :], lj)
            nf = (base + j).astype(jnp.float32)
            rec = jnp.where(iota == 0, m16, 0.0)
            rec = jnp.where(iota == 1, nf, rec)
            rec = jnp.where(iota == 2, cl16, rec)
            rec = jnp.where(iota == 3, x1c, rec)
            rec = jnp.where(iota == 4, y1c, rec)
            rec = jnp.where(iota == 5, x2c, rec)
            rec = jnp.where(iota == 6, y2c, rec)
            rec = jnp.where(iota == 7, ac, rec)
            rec_v[...] = rec
        pltpu.sync_copy(rec_v.at[pl.ds(0, 8)],
                        shared.at[sel, tid >> 1, pl.ds((tid & 1) * 8, 8)])
        plsc.subcore_barrier()
        pltpu.sync_copy(shared.at[sel], tab_v)

        zero16 = jnp.zeros((16,), jnp.int32)
        one16 = jnp.full((16,), 1, jnp.int32)
        eight16 = jnp.full((16,), 8, jnp.int32)
        vals = jnp.zeros((16,), jnp.float32)
        nvec = jnp.zeros((16,), jnp.float32)
        for r in range(NT // 2):
            rowt = tab_v[r, :]
            vals = jnp.where(iota == 2 * r, _g16(rowt, zero16), vals)
            vals = jnp.where(iota == 2 * r + 1, _g16(rowt, eight16), vals)
            nvec = jnp.where(iota == 2 * r, _g16(rowt, one16), nvec)
            nvec = jnp.where(iota == 2 * r + 1, _g16(rowt, eight16 + 1), nvec)
        gm16 = _bmax(vals, iota)
        nm = vals == gm16
        nstar16 = _bmin(jnp.where(nm, nvec, 1e9), iota)
        tcand = jnp.where(nm & (nvec == nstar16), fiota, 64.0)
        tstar = _smin(tcand, iota).astype(jnp.int32)
        wrec = tab_v[tstar >> 1, :]
        off16 = jnp.full((16,), (tstar & 1) * 8, jnp.int32)
        nstar16f = _g16(wrec, off16 + 1)
        cst16 = _g16(wrec, off16 + 2)
        cstar = _scal(cst16, iota).astype(jnp.int32)
        cstar_f = cst16
        X1 = _g16(wrec, off16 + 3)
        Y1 = _g16(wrec, off16 + 4)
        X2 = _g16(wrec, off16 + 5)
        Y2 = _g16(wrec, off16 + 6)
        A = _g16(wrec, off16 + 7)

        anyrs = []
        for g in range(NG):
            pv = blk[cstar * NG + g, :]
            x1v = blk[(C + cstar) * NG + g, :]
            y1v = blk[(2 * C + cstar) * NG + g, :]
            x2v = blk[(3 * C + cstar) * NG + g, :]
            y2v = blk[(4 * C + cstar) * NG + g, :]
            av = blk[(5 * C + cstar) * NG + g, :]
            iw = jnp.maximum(jnp.minimum(X2, x2v) - jnp.maximum(X1, x1v)
                             + 1.0, 0.0)
            ih = jnp.maximum(jnp.minimum(Y2, y2v) - jnp.maximum(Y1, y1v)
                             + 1.0, 0.0)
            inters = iw * ih
            union = (-inters + av) + A
            ovl = (inters / union) >= 0.5
            blk[cstar * NG + g, :] = jnp.where(ovl, 0.0, pv)
            cm = cmax2[g, :]
            cg = carg2[g, :]
            gidf = fiota + (base + g * 16).astype(jnp.float32)
            killm = gidf == nstar16f
            resc = ovl & (cg == cstar_f) & (cm > -0.5) & (gidf != nstar16f)
            anyr = _smax(jnp.where(resc, 1.0, 0.0), iota)
            anyrs.append(anyr)

            @pl.when(anyr > 0.5)
            def _(g=g, resc=resc, cm=cm, cg=cg):
                nmax = blk[g, :]
                narg = jnp.zeros((16,), jnp.float32)
                for c in range(1, C):
                    v = blk[c * NG + g, :]
                    upd = v > nmax
                    nmax = jnp.where(upd, v, nmax)
                    narg = jnp.where(upd, float(c), narg)
                cmax2[g, :] = jnp.where(resc, nmax, cm)
                carg2[g, :] = jnp.where(resc, narg, cg)

            cmk = cmax2[g, :]
            cmax2[g, :] = jnp.where(killm, -1.0, cmk)
            pdv = preds2[g, :]
            preds2[g, :] = jnp.where(killm, cstar_f, pdv)

        ns = _scal(nstar16f, iota)
        basef = base.astype(jnp.float32)
        kill_mine = (ns >= basef) & (ns < basef + float(RPT))
        dmax = jnp.maximum(jnp.maximum(anyrs[0], anyrs[1]),
                           jnp.maximum(anyrs[2], anyrs[3]))
        dflag = jnp.where(kill_mine, 1.0, dmax)
        dirty[0, :] = jnp.full((16,), dflag, jnp.float32)
        return carry

    lax.fori_loop(0, N, body, jnp.int32(0))
    for g in range(NG):
        predsi[g, :] = preds2[g, :].astype(jnp.int32)
    pltpu.sync_copy(predsi, out_hbm.at[tid])


_sc_call = functools.partial(
    pl.kernel,
    out_type=jax.ShapeDtypeStruct((NT, NG, 16), jnp.int32),
    mesh=plsc.VectorSubcoreMesh(core_axis_name="c", subcore_axis_name="s",
                                num_cores=1, num_subcores=NT),
    compiler_params=pltpu.CompilerParams(use_tc_tiling_on_sc=False),
    scratch_types=[
        pltpu.VMEM((RB * NG, 16), jnp.float32),
        pltpu.VMEM((NG, 16), jnp.float32),
        pltpu.VMEM((NG, 16), jnp.float32),
        pltpu.VMEM((NG, 16), jnp.float32),
        pltpu.VMEM((NG, 16), jnp.int32),
        pltpu.VMEM((16,), jnp.float32),
        pltpu.VMEM((NT // 2, 16), jnp.float32),
        pltpu.VMEM((1, 16), jnp.float32),
        pltpu.VMEM_SHARED((2, NT // 2, 16), jnp.float32),
    ],
)(_sc_greedy)


def kernel(obj_dists1, obj_feats, obj_labels, box_priors, boxes_per_cls, W, b):
    del obj_labels, box_priors
    b2 = b[None, :]
    x1 = boxes_per_cls[:, :, 0]
    y1 = boxes_per_cls[:, :, 1]
    x2 = boxes_per_cls[:, :, 2]
    y2 = boxes_per_cls[:, :, 3]
    d2, stk = pl.pallas_call(
        _prep_kernel,
        out_shape=(
            jax.ShapeDtypeStruct((N, C), jnp.float32),
            jax.ShapeDtypeStruct((NT, RB, RPT), jnp.float32),
        ),
    )(obj_dists1, obj_feats, W, b2, x1, y1, x2, y2)
    preds = _sc_call(stk.reshape(NT, RB * NG, 16))
    return d2, preds.reshape(NP)[:N]
